# baseline (speedup 1.0000x reference)
import functools
import math

import jax
import jax.numpy as jnp
from jax import lax
from jax.experimental import pallas as pl
from jax.experimental.pallas import tpu as pltpu
from jax.experimental.pallas import tpu_sc as plsc

N_NODES = 10000
N_EDGES = 320000
HIDDEN = 128
N_HEADS = 8
HEAD_DIM = HIDDEN // N_HEADS

NC = 2
NS = 16
NW = NC * NS
PER_W = N_EDGES // NW
C = 16
CHUNKS = PER_W // C
ROW_BLK = 624
ROW_TAIL = N_NODES - ROW_BLK * NS


def _proj_body(x_ref, wq_ref, wk_ref, wv_ref, q_ref, k_ref, v_ref):
    xb = x_ref[...]
    q_ref[...] = jnp.dot(xb, wq_ref[...], preferred_element_type=jnp.float32)
    k_ref[...] = jnp.dot(xb, wk_ref[...], preferred_element_type=jnp.float32)
    v_ref[...] = jnp.dot(xb, wv_ref[...], preferred_element_type=jnp.float32)


def _project(x, wq_t, wk_t, wv_t):
    blk = 400
    grid = (N_NODES // blk,)
    out = jax.ShapeDtypeStruct((N_NODES, HIDDEN), jnp.float32)
    w_spec = pl.BlockSpec((HIDDEN, HIDDEN), lambda i: (0, 0))
    return pl.pallas_call(
        _proj_body,
        grid=grid,
        in_specs=[pl.BlockSpec((blk, HIDDEN), lambda i: (i, 0)),
                  w_spec, w_spec, w_spec],
        out_specs=[pl.BlockSpec((blk, HIDDEN), lambda i: (i, 0))] * 3,
        out_shape=[out, out, out],
    )(x, wq_t, wk_t, wv_t)


def _combine_body(p_ref, o_ref):
    o_ref[...] = p_ref[0] + p_ref[1]


def _combine(partials):
    blk = 400
    return pl.pallas_call(
        _combine_body,
        grid=(N_NODES // blk,),
        in_specs=[pl.BlockSpec((NC, blk, HIDDEN), lambda i: (0, i, 0))],
        out_specs=pl.BlockSpec((blk, HIDDEN), lambda i: (i, 0)),
        out_shape=jax.ShapeDtypeStruct((N_NODES, HIDDEN), jnp.float32),
    )(partials)


def _dyn_splat(vec16, lane):
    idx = jnp.full((16,), lane, dtype=jnp.int32)
    return lax.gather(
        vec16, idx[:, None],
        lax.GatherDimensionNumbers(offset_dims=(), collapsed_slice_dims=(0,),
                                   start_index_map=(0,)),
        (1,), mode=lax.GatherScatterMode.PROMISE_IN_BOUNDS)


def _edge_kernel(q, k, v, w_ij, cut, src, dst):
    mesh = plsc.VectorSubcoreMesh(core_axis_name="c", subcore_axis_name="s",
                                  num_cores=NC, num_subcores=NS)

    @functools.partial(
        pl.kernel,
        out_type=jax.ShapeDtypeStruct((NC, N_NODES, HIDDEN), jnp.float32),
        mesh=mesh,
        scratch_types=[
            [pltpu.VMEM((C,), jnp.int32)] * 2,
            [pltpu.VMEM((C,), jnp.int32)] * 2,
            [pltpu.VMEM((C,), jnp.int32)] * 2,
            [pltpu.VMEM((C,), jnp.float32)] * 2,
            [pltpu.VMEM((C, HIDDEN), jnp.float32)] * 2,
            [pltpu.VMEM((C, HIDDEN), jnp.float32)] * 2,
            [pltpu.VMEM((C, HIDDEN), jnp.float32)] * 2,
            [pltpu.VMEM((C, HIDDEN), jnp.float32)] * 2,
            pltpu.VMEM_SHARED((N_NODES, HIDDEN), jnp.float32),
            [pltpu.SemaphoreType.DMA] * 2,
            [pltpu.SemaphoreType.DMA] * 2,
            [pltpu.SemaphoreType.DMA] * 2,
        ],
        compiler_params=pltpu.CompilerParams(needs_layout_passes=False),
    )
    def k_fn(q_hbm, k_hbm, v_hbm, w_hbm, cut_hbm, src_hbm, dst_hbm,
             out_hbm,
             isrc, idst, sdst, cb, qb, kb, vb, wb, acc,
             lsem, gsem, ssem):
        c = lax.axis_index("c")
        s = lax.axis_index("s")
        wid = c * NS + s
        z16 = jnp.zeros((16,), jnp.float32)

        def loads_io(t, p):
            base = wid * PER_W + t * C
            return [(src_hbm.at[pl.ds(base, C)], isrc[p], lsem[p]),
                    (dst_hbm.at[pl.ds(base, C)], idst[p], lsem[p]),
                    (cut_hbm.at[pl.ds(base, C)], cb[p], lsem[p])]

        def gathers_io(t, p):
            base = wid * PER_W + t * C
            return [(q_hbm.at[idst[p]], qb[p], gsem[p]),
                    (k_hbm.at[isrc[p]], kb[p], gsem[p]),
                    (v_hbm.at[isrc[p]], vb[p], gsem[p]),
                    (w_hbm.at[pl.ds(base, C), :], wb[p], gsem[p])]

        def issue(ios):
            for a, b, sem in ios:
                pltpu.async_copy(a, b, sem)

        def drain(ios):
            for a, b, sem in ios:
                pltpu.make_async_copy(a, b, sem).wait()

        def scatter_io(p):
            return (wb[p], acc.at[sdst[p]], ssem[p])

        def compute(p):
            cvec = cb[p][...] * (1.0 / math.sqrt(HEAD_DIM))

            def edge(j, carry):
                cut_splat = _dyn_splat(cvec, j)
                for h in range(N_HEADS):
                    sl = pl.ds(h * HEAD_DIM, HEAD_DIM)
                    tt = qb[p][j, sl] * wb[p][j, sl] * kb[p][j, sl]
                    cs = plsc.cumsum(tt)
                    ssp = _dyn_splat(cs, 15)
                    wb[p][j, sl] = vb[p][j, sl] * (ssp * cut_splat)
                return carry
            lax.fori_loop(0, C, edge, None)

        def zero_wb(i, carry):
            wb[0][i // 8, pl.ds((i % 8) * 16, 16)] = z16
            return carry
        lax.fori_loop(0, C * 8, zero_wb, None)

        def zero_ios(i):
            return [(wb[0][...], acc.at[pl.ds(s * ROW_BLK + i * 16, 16), :],
                     gsem[0])]

        def zero_fire(i, carry):
            pltpu.async_copy(wb[0], acc.at[pl.ds(s * ROW_BLK + i * 16, 16), :],
                             gsem[0])
            return carry
        lax.fori_loop(0, ROW_BLK // 16, zero_fire, None)

        @pl.when(s == 0)
        def _zero_tail():
            pltpu.async_copy(wb[0], acc.at[pl.ds(ROW_BLK * NS, ROW_TAIL), :],
                             gsem[0])

        def zero_drain(i, carry):
            pltpu.make_async_copy(
                wb[0], acc.at[pl.ds(s * ROW_BLK + i * 16, 16), :],
                gsem[0]).wait()
            return carry
        lax.fori_loop(0, ROW_BLK // 16, zero_drain, None)

        @pl.when(s == 0)
        def _zero_tail_drain():
            pltpu.make_async_copy(
                wb[0], acc.at[pl.ds(ROW_BLK * NS, ROW_TAIL), :],
                gsem[0]).wait()
        plsc.subcore_barrier()

        issue(loads_io(0, 0))
        issue(loads_io(1, 1))
        drain(loads_io(0, 0))
        issue(gathers_io(0, 0))

        def two_chunks(t2, carry):
            for b in range(2):
                nb = 1 - b
                t = t2 * 2 + b
                if b == 0:
                    @pl.when(t2 >= 1)
                    def _w():
                        a, d, sem = scatter_io(nb)
                        pltpu.make_async_copy(a, d, sem).wait()
                else:
                    a, d, sem = scatter_io(nb)
                    pltpu.make_async_copy(a, d, sem).wait()
                drain(loads_io(t + 1, nb))
                issue(gathers_io(t + 1, nb))
                drain(gathers_io(t, b))
                sdst[b][...] = idst[b][...]
                compute(b)
                if b == 0:
                    issue(loads_io(t + 2, b))
                else:
                    @pl.when(t2 < (CHUNKS - 3) // 2)
                    def _l():
                        issue(loads_io(t + 2, b))
                a, d, sem = scatter_io(b)
                pltpu.async_copy(a, d, sem, add=True)
            return carry
        lax.fori_loop(0, (CHUNKS - 1) // 2, two_chunks, None)

        a, d, sem = scatter_io(1)
        pltpu.make_async_copy(a, d, sem).wait()
        drain(gathers_io(CHUNKS - 1, 0))
        sdst[0][...] = idst[0][...]
        compute(0)
        a, d, sem = scatter_io(0)
        pltpu.async_copy(a, d, sem, add=True)
        pltpu.make_async_copy(a, d, sem).wait()

        plsc.subcore_barrier()
        pltpu.sync_copy(acc.at[pl.ds(s * ROW_BLK, ROW_BLK), :],
                        out_hbm.at[c, pl.ds(s * ROW_BLK, ROW_BLK), :])

        @pl.when(s == 0)
        def _write_tail():
            pltpu.sync_copy(acc.at[pl.ds(ROW_BLK * NS, ROW_TAIL), :],
                            out_hbm.at[c, pl.ds(ROW_BLK * NS, ROW_TAIL), :])

    return k_fn(q, k, v, w_ij, cut, src, dst)


def kernel(x, w_ij, edge_index, cutoff, Wq, Wk, Wv):
    src = edge_index[0].astype(jnp.int32)
    dst = edge_index[1].astype(jnp.int32)
    cut = cutoff.reshape(-1)
    q, k, v = _project(x, Wq.T, Wk.T, Wv.T)
    partials = _edge_kernel(q, k, v, w_ij, cut, src, dst)
    return _combine(partials)


# speedup vs baseline: 1.0614x; 1.0614x over previous
import functools
import math

import jax
import jax.numpy as jnp
from jax import lax
from jax.experimental import pallas as pl
from jax.experimental.pallas import tpu as pltpu
from jax.experimental.pallas import tpu_sc as plsc

N_NODES = 10000
N_EDGES = 320000
HIDDEN = 128
N_HEADS = 8
HEAD_DIM = HIDDEN // N_HEADS

NC = 2
NS = 16
NW = NC * NS
PER_W = N_EDGES // NW
C = 48
CHUNKS = PER_W // C
TAIL = PER_W - CHUNKS * C
ROW_BLK = 624
ROW_TAIL = N_NODES - ROW_BLK * NS
INV_SQRT_D = 1.0 / math.sqrt(HEAD_DIM)


def _proj_body(x_ref, wq_ref, wk_ref, wv_ref, q_ref, k_ref, v_ref):
    xb = x_ref[...]
    q_ref[...] = jnp.dot(xb, wq_ref[...], preferred_element_type=jnp.float32)
    k_ref[...] = jnp.dot(xb, wk_ref[...], preferred_element_type=jnp.float32)
    v_ref[...] = jnp.dot(xb, wv_ref[...], preferred_element_type=jnp.float32)


def _project(x, wq_t, wk_t, wv_t):
    blk = 400
    out = jax.ShapeDtypeStruct((N_NODES, HIDDEN), jnp.float32)
    w_spec = pl.BlockSpec((HIDDEN, HIDDEN), lambda i: (0, 0))
    return pl.pallas_call(
        _proj_body,
        grid=(N_NODES // blk,),
        in_specs=[pl.BlockSpec((blk, HIDDEN), lambda i: (i, 0)),
                  w_spec, w_spec, w_spec],
        out_specs=[pl.BlockSpec((blk, HIDDEN), lambda i: (i, 0))] * 3,
        out_shape=[out, out, out],
    )(x, wq_t, wk_t, wv_t)


def _combine_body(p_ref, o_ref):
    o_ref[...] = p_ref[0] + p_ref[1]


def _combine(partials):
    blk = 400
    return pl.pallas_call(
        _combine_body,
        grid=(N_NODES // blk,),
        in_specs=[pl.BlockSpec((NC, blk, HIDDEN), lambda i: (0, i, 0))],
        out_specs=pl.BlockSpec((blk, HIDDEN), lambda i: (i, 0)),
        out_shape=jax.ShapeDtypeStruct((N_NODES, HIDDEN), jnp.float32),
    )(partials)


def _dyn_splat(vec16, lane):
    idx = jnp.full((16,), lane, dtype=jnp.int32)
    return lax.gather(
        vec16, idx[:, None],
        lax.GatherDimensionNumbers(offset_dims=(), collapsed_slice_dims=(0,),
                                   start_index_map=(0,)),
        (1,), mode=lax.GatherScatterMode.PROMISE_IN_BOUNDS)


def _edge_kernel(q, k, v, w_ij, cut, src, dst):
    mesh = plsc.VectorSubcoreMesh(core_axis_name="c", subcore_axis_name="s",
                                  num_cores=NC, num_subcores=NS)

    @functools.partial(
        pl.kernel,
        out_type=jax.ShapeDtypeStruct((NC, N_NODES, HIDDEN), jnp.float32),
        mesh=mesh,
        scratch_types=[
            [pltpu.VMEM((C,), jnp.int32)] * 2,
            [pltpu.VMEM((C,), jnp.int32)] * 2,
            [pltpu.VMEM((C,), jnp.int32)] * 2,
            [pltpu.VMEM((C,), jnp.float32)] * 2,
            [pltpu.VMEM((C, HIDDEN), jnp.float32)] * 2,
            [pltpu.VMEM((C, HIDDEN), jnp.float32)] * 2,
            [pltpu.VMEM((C, HIDDEN), jnp.float32)] * 2,
            [pltpu.VMEM((C, HIDDEN), jnp.float32)] * 2,
            pltpu.VMEM((16,), jnp.int32),
            pltpu.VMEM_SHARED((N_NODES, HIDDEN), jnp.float32),
            [pltpu.SemaphoreType.DMA] * 2,
            [pltpu.SemaphoreType.DMA] * 2,
            [pltpu.SemaphoreType.DMA] * 2,
        ],
        compiler_params=pltpu.CompilerParams(needs_layout_passes=False),
    )
    def k_fn(q_hbm, k_hbm, v_hbm, w_hbm, cut_hbm, src_hbm, dst_hbm,
             out_hbm,
             isrc, idst, scdst, cb, qb, kb, vb, wb, tdst, acc,
             lsem, gsem, ssem):
        c = lax.axis_index("c")
        s = lax.axis_index("s")
        wid = c * NS + s
        z16 = jnp.zeros((16,), jnp.float32)

        def loads_io(t, p, n=C):
            base = wid * PER_W + t * C
            return [(src_hbm.at[pl.ds(base, n)], isrc[p], lsem[p]),
                    (dst_hbm.at[pl.ds(base, n)], idst[p], lsem[p]),
                    (cut_hbm.at[pl.ds(base, n)], cb[p], lsem[p]),
                    (w_hbm.at[pl.ds(base, n), :], wb[p], lsem[p])]

        def gathers_io(p):
            return [(q_hbm.at[idst[p]], qb[p], gsem[p]),
                    (k_hbm.at[isrc[p]], kb[p], gsem[p]),
                    (v_hbm.at[isrc[p]], vb[p], gsem[p])]

        def issue(ios):
            for a, b, sem in ios:
                pltpu.async_copy(a, b, sem)

        def drain(ios):
            for a, b, sem in ios:
                pltpu.make_async_copy(a, b, sem).wait()

        def scatter_io(p):
            return (wb[p], acc.at[scdst[p]], ssem[p])

        def save_scatter_idx(p):
            for g in range(C // 16):
                sl = pl.ds(g * 16, 16)
                scdst[p][sl] = idst[p][sl]

        def compute_group(p, g):
            cvec = cb[p][pl.ds(g * 16, 16)] * INV_SQRT_D

            def edge(j, carry):
                e = g * 16 + j
                cut_splat = _dyn_splat(cvec, j)
                for h in range(N_HEADS):
                    sl = pl.ds(h * HEAD_DIM, HEAD_DIM)
                    tt = qb[p][e, sl] * wb[p][e, sl] * kb[p][e, sl]
                    cs = plsc.cumsum(tt)
                    ssp = _dyn_splat(cs, 15)
                    wb[p][e, sl] = vb[p][e, sl] * (ssp * cut_splat)
                return carry
            lax.fori_loop(0, 16, edge, None)

        def compute(p):
            def group(g, carry):
                compute_group(p, g)
                return carry
            lax.fori_loop(0, C // 16, group, None)

        def zero_wb(i, carry):
            wb[0][i // 8, pl.ds((i % 8) * 16, 16)] = z16
            return carry
        lax.fori_loop(0, C * 8, zero_wb, None)

        def zero_fire(i, carry):
            pltpu.async_copy(wb[0].at[pl.ds(0, 16), :],
                             acc.at[pl.ds(s * ROW_BLK + i * 16, 16), :],
                             gsem[0])
            return carry
        lax.fori_loop(0, ROW_BLK // 16, zero_fire, None)

        @pl.when(s == 0)
        def _zero_tail():
            pltpu.async_copy(wb[0].at[pl.ds(0, ROW_TAIL), :],
                             acc.at[pl.ds(ROW_BLK * NS, ROW_TAIL), :],
                             gsem[0])

        def zero_drain(i, carry):
            pltpu.make_async_copy(
                wb[0].at[pl.ds(0, 16), :],
                acc.at[pl.ds(s * ROW_BLK + i * 16, 16), :],
                gsem[0]).wait()
            return carry
        lax.fori_loop(0, ROW_BLK // 16, zero_drain, None)

        @pl.when(s == 0)
        def _zero_tail_drain():
            pltpu.make_async_copy(
                wb[0].at[pl.ds(0, ROW_TAIL), :],
                acc.at[pl.ds(ROW_BLK * NS, ROW_TAIL), :],
                gsem[0]).wait()
        plsc.subcore_barrier()

        issue(loads_io(0, 0))
        issue(loads_io(1, 1))
        drain(loads_io(0, 0))
        save_scatter_idx(0)
        issue(gathers_io(0))

        def two_chunks(t2, carry):
            for b in range(2):
                nb = 1 - b
                t = t2 * 2 + b
                if b == 0:
                    @pl.when(t2 >= 1)
                    def _w():
                        a, d, sem = scatter_io(nb)
                        pltpu.make_async_copy(a, d, sem).wait()
                else:
                    a, d, sem = scatter_io(nb)
                    pltpu.make_async_copy(a, d, sem).wait()

                if b == 0:
                    drain(loads_io(t + 1, nb))
                    save_scatter_idx(nb)
                    issue(gathers_io(nb))
                else:
                    @pl.when(t2 < CHUNKS // 2 - 1)
                    def _g():
                        drain(loads_io(t + 1, nb))
                        save_scatter_idx(nb)
                        issue(gathers_io(nb))

                drain(gathers_io(b))
                compute(b)

                @pl.when(t2 < CHUNKS // 2 - 1)
                def _l():
                    issue(loads_io(t + 2, b))

                a, d, sem = scatter_io(b)
                pltpu.async_copy(a, d, sem, add=True)
            return carry
        lax.fori_loop(0, CHUNKS // 2, two_chunks, None)

        a, d, sem = scatter_io(1)
        pltpu.make_async_copy(a, d, sem).wait()

        tbase = wid * PER_W + CHUNKS * C
        pltpu.sync_copy(src_hbm.at[pl.ds(tbase, TAIL)],
                        isrc[0].at[pl.ds(0, TAIL)])
        pltpu.sync_copy(dst_hbm.at[pl.ds(tbase, TAIL)],
                        idst[0].at[pl.ds(0, TAIL)])
        pltpu.sync_copy(cut_hbm.at[pl.ds(tbase, TAIL)],
                        cb[0].at[pl.ds(0, TAIL)])
        pltpu.sync_copy(w_hbm.at[pl.ds(tbase, TAIL), :],
                        wb[0].at[pl.ds(0, TAIL), :])
        tdst[...] = idst[0][pl.ds(0, TAIL)]
        cp0 = pltpu.async_copy(q_hbm.at[tdst], qb[0].at[pl.ds(0, TAIL), :],
                               gsem[0])
        cp1 = pltpu.async_copy(k_hbm.at[isrc[0].at[pl.ds(0, TAIL)]],
                               kb[0].at[pl.ds(0, TAIL), :], gsem[0])
        cp2 = pltpu.async_copy(v_hbm.at[isrc[0].at[pl.ds(0, TAIL)]],
                               vb[0].at[pl.ds(0, TAIL), :], gsem[0])
        cp0.wait()
        cp1.wait()
        cp2.wait()
        compute_group(0, 0)
        pltpu.sync_copy(wb[0].at[pl.ds(0, TAIL), :], acc.at[tdst], add=True)

        plsc.subcore_barrier()
        pltpu.sync_copy(acc.at[pl.ds(s * ROW_BLK, ROW_BLK), :],
                        out_hbm.at[c, pl.ds(s * ROW_BLK, ROW_BLK), :])

        @pl.when(s == 0)
        def _write_tail():
            pltpu.sync_copy(acc.at[pl.ds(ROW_BLK * NS, ROW_TAIL), :],
                            out_hbm.at[c, pl.ds(ROW_BLK * NS, ROW_TAIL), :])

    return k_fn(q, k, v, w_ij, cut, src, dst)


def kernel(x, w_ij, edge_index, cutoff, Wq, Wk, Wv):
    src = edge_index[0].astype(jnp.int32)
    dst = edge_index[1].astype(jnp.int32)
    cut = cutoff.reshape(-1)
    q, k, v = _project(x, Wq.T, Wk.T, Wv.T)
    partials = _edge_kernel(q, k, v, w_ij, cut, src, dst)
    return _combine(partials)


# speedup vs baseline: 2.0702x; 1.9505x over previous
import functools
import math

import jax
import jax.numpy as jnp
from jax import lax
from jax.experimental import pallas as pl
from jax.experimental.pallas import tpu as pltpu
from jax.experimental.pallas import tpu_sc as plsc

N_NODES = 10000
N_EDGES = 320000
HIDDEN = 128
N_HEADS = 8
HEAD_DIM = HIDDEN // N_HEADS

NC = 2
NS = 16
NW = NC * NS
PER_W = N_EDGES // NW
C = 32
CHUNKS = PER_W // C
TAIL = PER_W - CHUNKS * C
ROW_BLK = 624
ROW_TAIL = N_NODES - ROW_BLK * NS
INV_SQRT_D = 1.0 / math.sqrt(HEAD_DIM)


def _proj_body(x_ref, wq_ref, wk_ref, wv_ref, q_ref, k_ref, v_ref):
    xb = x_ref[...]
    q_ref[...] = jnp.dot(xb, wq_ref[...], preferred_element_type=jnp.float32)
    k_ref[...] = jnp.dot(xb, wk_ref[...], preferred_element_type=jnp.float32)
    v_ref[...] = jnp.dot(xb, wv_ref[...], preferred_element_type=jnp.float32)


def _project(x, wq_t, wk_t, wv_t):
    blk = 400
    out = jax.ShapeDtypeStruct((N_NODES, HIDDEN), jnp.float32)
    w_spec = pl.BlockSpec((HIDDEN, HIDDEN), lambda i: (0, 0))
    return pl.pallas_call(
        _proj_body,
        grid=(N_NODES // blk,),
        in_specs=[pl.BlockSpec((blk, HIDDEN), lambda i: (i, 0)),
                  w_spec, w_spec, w_spec],
        out_specs=[pl.BlockSpec((blk, HIDDEN), lambda i: (i, 0))] * 3,
        out_shape=[out, out, out],
    )(x, wq_t, wk_t, wv_t)


def _combine_body(p_ref, o_ref):
    o_ref[...] = p_ref[0] + p_ref[1]


def _combine(partials):
    blk = 400
    return pl.pallas_call(
        _combine_body,
        grid=(N_NODES // blk,),
        in_specs=[pl.BlockSpec((NC, blk, HIDDEN), lambda i: (0, i, 0))],
        out_specs=pl.BlockSpec((blk, HIDDEN), lambda i: (i, 0)),
        out_shape=jax.ShapeDtypeStruct((N_NODES, HIDDEN), jnp.float32),
    )(partials)


def _dyn_splat(vec16, lane):
    idx = jnp.full((16,), lane, dtype=jnp.int32)
    return lax.gather(
        vec16, idx[:, None],
        lax.GatherDimensionNumbers(offset_dims=(), collapsed_slice_dims=(0,),
                                   start_index_map=(0,)),
        (1,), mode=lax.GatherScatterMode.PROMISE_IN_BOUNDS)


def _edge_kernel(q, k, v, w_ij, cut, src, dst):
    mesh = plsc.VectorSubcoreMesh(core_axis_name="c", subcore_axis_name="s",
                                  num_cores=NC, num_subcores=NS)

    @functools.partial(
        pl.kernel,
        out_type=jax.ShapeDtypeStruct((NC, N_NODES, HIDDEN), jnp.float32),
        mesh=mesh,
        scratch_types=[
            [pltpu.VMEM((C,), jnp.int32)] * 2,
            [pltpu.VMEM((C,), jnp.int32)] * 2,
            [pltpu.VMEM((C,), jnp.int32)] * 2,
            [pltpu.VMEM((C,), jnp.float32)] * 2,
            [pltpu.VMEM((C, HIDDEN), jnp.float32)] * 2,
            [pltpu.VMEM((C, HIDDEN), jnp.float32)] * 2,
            [pltpu.VMEM((C, HIDDEN), jnp.float32)] * 2,
            [pltpu.VMEM((C, HIDDEN), jnp.float32)] * 2,
            pltpu.VMEM((16,), jnp.int32),
            pltpu.VMEM_SHARED((N_NODES, HIDDEN), jnp.float32),
            [pltpu.SemaphoreType.DMA] * 2,
            [pltpu.SemaphoreType.DMA] * 2,
            [pltpu.SemaphoreType.DMA] * 2,
        ],
        compiler_params=pltpu.CompilerParams(needs_layout_passes=False),
    )
    def k_fn(q_hbm, k_hbm, v_hbm, w_hbm, cut_hbm, src_hbm, dst_hbm,
             out_hbm,
             isrc, idst, scdst, cb, qb, kb, vb, wb, tdst, acc,
             lsem, gsem, ssem):
        c = lax.axis_index("c")
        s = lax.axis_index("s")
        wid = c * NS + s
        z16 = jnp.zeros((16,), jnp.float32)

        def loads_io(t, p, n=C):
            base = wid * PER_W + t * C
            return [(src_hbm.at[pl.ds(base, n)], isrc[p], lsem[p]),
                    (dst_hbm.at[pl.ds(base, n)], idst[p], lsem[p]),
                    (cut_hbm.at[pl.ds(base, n)], cb[p], lsem[p]),
                    (w_hbm.at[pl.ds(base, n), :], wb[p], lsem[p])]

        def gathers_io(p):
            return [(q_hbm.at[idst[p]], qb[p], gsem[p]),
                    (k_hbm.at[isrc[p]], kb[p], gsem[p]),
                    (v_hbm.at[isrc[p]], vb[p], gsem[p])]

        def issue(ios):
            for a, b, sem in ios:
                pltpu.async_copy(a, b, sem)

        def drain(ios):
            for a, b, sem in ios:
                pltpu.make_async_copy(a, b, sem).wait()

        def scatter_io(p):
            return (wb[p], acc.at[scdst[p]], ssem[p])

        def save_scatter_idx(p):
            for g in range(C // 16):
                sl = pl.ds(g * 16, 16)
                scdst[p][sl] = idst[p][sl]

        def compute_group(p, g):
            cvec = cb[p][pl.ds(g * 16, 16)] * INV_SQRT_D
            for j in range(16):
                e = g * 16 + j
                cut_splat = _dyn_splat(cvec, j)
                for h in range(N_HEADS):
                    sl = pl.ds(h * HEAD_DIM, HEAD_DIM)
                    tt = qb[p][e, sl] * wb[p][e, sl] * kb[p][e, sl]
                    cs = plsc.cumsum(tt)
                    ssp = _dyn_splat(cs, 15)
                    wb[p][e, sl] = vb[p][e, sl] * (ssp * cut_splat)

        def compute(p):
            def group(g, carry):
                compute_group(p, g)
                return carry
            lax.fori_loop(0, C // 16, group, None)

        def zero_wb(i, carry):
            wb[0][i // 8, pl.ds((i % 8) * 16, 16)] = z16
            return carry
        lax.fori_loop(0, C * 8, zero_wb, None)

        def zero_fire(i, carry):
            pltpu.async_copy(wb[0].at[pl.ds(0, 16), :],
                             acc.at[pl.ds(s * ROW_BLK + i * 16, 16), :],
                             gsem[0])
            return carry
        lax.fori_loop(0, ROW_BLK // 16, zero_fire, None)

        @pl.when(s == 0)
        def _zero_tail():
            pltpu.async_copy(wb[0].at[pl.ds(0, ROW_TAIL), :],
                             acc.at[pl.ds(ROW_BLK * NS, ROW_TAIL), :],
                             gsem[0])

        def zero_drain(i, carry):
            pltpu.make_async_copy(
                wb[0].at[pl.ds(0, 16), :],
                acc.at[pl.ds(s * ROW_BLK + i * 16, 16), :],
                gsem[0]).wait()
            return carry
        lax.fori_loop(0, ROW_BLK // 16, zero_drain, None)

        @pl.when(s == 0)
        def _zero_tail_drain():
            pltpu.make_async_copy(
                wb[0].at[pl.ds(0, ROW_TAIL), :],
                acc.at[pl.ds(ROW_BLK * NS, ROW_TAIL), :],
                gsem[0]).wait()
        plsc.subcore_barrier()

        issue(loads_io(0, 0))
        issue(loads_io(1, 1))
        drain(loads_io(0, 0))
        save_scatter_idx(0)
        issue(gathers_io(0))

        def two_chunks(t2, carry):
            for b in range(2):
                nb = 1 - b
                t = t2 * 2 + b
                if b == 0:
                    @pl.when(t2 >= 1)
                    def _w():
                        a, d, sem = scatter_io(nb)
                        pltpu.make_async_copy(a, d, sem).wait()
                else:
                    a, d, sem = scatter_io(nb)
                    pltpu.make_async_copy(a, d, sem).wait()

                if b == 0:
                    drain(loads_io(t + 1, nb))
                    save_scatter_idx(nb)
                    issue(gathers_io(nb))
                else:
                    @pl.when(t2 < CHUNKS // 2 - 1)
                    def _g():
                        drain(loads_io(t + 1, nb))
                        save_scatter_idx(nb)
                        issue(gathers_io(nb))

                drain(gathers_io(b))
                compute(b)

                @pl.when(t2 < CHUNKS // 2 - 1)
                def _l():
                    issue(loads_io(t + 2, b))

                a, d, sem = scatter_io(b)
                pltpu.async_copy(a, d, sem, add=True)
            return carry
        lax.fori_loop(0, CHUNKS // 2, two_chunks, None)

        a, d, sem = scatter_io(1)
        pltpu.make_async_copy(a, d, sem).wait()

        tbase = wid * PER_W + CHUNKS * C
        pltpu.sync_copy(src_hbm.at[pl.ds(tbase, TAIL)],
                        isrc[0].at[pl.ds(0, TAIL)])
        pltpu.sync_copy(dst_hbm.at[pl.ds(tbase, TAIL)],
                        idst[0].at[pl.ds(0, TAIL)])
        pltpu.sync_copy(cut_hbm.at[pl.ds(tbase, TAIL)],
                        cb[0].at[pl.ds(0, TAIL)])
        pltpu.sync_copy(w_hbm.at[pl.ds(tbase, TAIL), :],
                        wb[0].at[pl.ds(0, TAIL), :])
        tdst[...] = idst[0][pl.ds(0, TAIL)]
        cp0 = pltpu.async_copy(q_hbm.at[tdst], qb[0].at[pl.ds(0, TAIL), :],
                               gsem[0])
        cp1 = pltpu.async_copy(k_hbm.at[isrc[0].at[pl.ds(0, TAIL)]],
                               kb[0].at[pl.ds(0, TAIL), :], gsem[0])
        cp2 = pltpu.async_copy(v_hbm.at[isrc[0].at[pl.ds(0, TAIL)]],
                               vb[0].at[pl.ds(0, TAIL), :], gsem[0])
        cp0.wait()
        cp1.wait()
        cp2.wait()
        compute_group(0, 0)
        pltpu.sync_copy(wb[0].at[pl.ds(0, TAIL), :], acc.at[tdst], add=True)

        plsc.subcore_barrier()
        pltpu.sync_copy(acc.at[pl.ds(s * ROW_BLK, ROW_BLK), :],
                        out_hbm.at[c, pl.ds(s * ROW_BLK, ROW_BLK), :])

        @pl.when(s == 0)
        def _write_tail():
            pltpu.sync_copy(acc.at[pl.ds(ROW_BLK * NS, ROW_TAIL), :],
                            out_hbm.at[c, pl.ds(ROW_BLK * NS, ROW_TAIL), :])

    return k_fn(q, k, v, w_ij, cut, src, dst)


def kernel(x, w_ij, edge_index, cutoff, Wq, Wk, Wv):
    src = edge_index[0].astype(jnp.int32)
    dst = edge_index[1].astype(jnp.int32)
    cut = cutoff.reshape(-1)
    q, k, v = _project(x, Wq.T, Wk.T, Wv.T)
    partials = _edge_kernel(q, k, v, w_ij, cut, src, dst)
    return _combine(partials)
